# baseline (speedup 1.0000x reference)
import jax
import jax.numpy as jnp
from jax.experimental import pallas as pl
from jax.experimental.pallas import tpu as pltpu

_MIB = 1024 * 1024


def _fused_kernel(x1_ref, x2_ref, w_ref, b_ref, o_ref):
    cout, k, _ = w_ref.shape
    w2d = w_ref[...].reshape(cout, k * 128)
    half = (k * 128) // 2
    s1 = jnp.sum(x1_ref[0], axis=0, keepdims=True)
    s2 = jnp.sum(x2_ref[0], axis=0, keepdims=True)
    dn = (((1,), (1,)), ((), ()))
    y = (jax.lax.dot_general(s1, w2d[:, :half], dn,
                             preferred_element_type=jnp.float32)
         + jax.lax.dot_general(s2, w2d[:, half:], dn,
                               preferred_element_type=jnp.float32))
    z = jnp.maximum(y + b_ref[...], 0.0)
    o_ref[0] = jnp.broadcast_to(z, o_ref.shape[1:]).astype(o_ref.dtype)


def kernel(x, conv_w, bn_gamma, bn_beta, bn_mean, bn_var, eps=1e-5):
    N, Cin, H, W = x.shape
    Cout = conv_w.shape[0]
    HW = H * W

    scale = (bn_gamma.astype(jnp.float32)
             / jnp.sqrt(bn_var.astype(jnp.float32) + eps))
    bias = bn_beta.astype(jnp.float32) - bn_mean.astype(jnp.float32) * scale
    beta = bias[None, :]
    wr = (conv_w.reshape(Cout, Cin // 128, 128).astype(jnp.float32)
          * (scale * (1.0 / HW))[:, None, None])

    xv = jnp.transpose(x, (0, 2, 3, 1)).reshape(N, HW, Cin)
    itemsize = jnp.dtype(x.dtype).itemsize
    ch = Cin // 2

    out = pl.pallas_call(
        _fused_kernel,
        out_shape=jax.ShapeDtypeStruct((N, HW, Cout), x.dtype),
        grid=(N,),
        in_specs=[
            pl.BlockSpec((1, HW, ch), lambda n: (n, 0, 0)),
            pl.BlockSpec((1, HW, ch), lambda n: (n, 0, 1)),
            pl.BlockSpec((Cout, Cin // 128, 128), lambda n: (0, 0, 0)),
            pl.BlockSpec((1, Cout), lambda n: (0, 0)),
        ],
        out_specs=pl.BlockSpec((1, HW, Cout), lambda n: (n, 0, 0)),
        compiler_params=pltpu.CompilerParams(
            dimension_semantics=("parallel",),
            vmem_limit_bytes=48 * _MIB),
        cost_estimate=pl.CostEstimate(
            flops=int(N * Cin * HW + 2 * N * Cin * Cout),
            transcendentals=0,
            bytes_accessed=int(N * Cin * HW * itemsize
                               + N * Cout * HW * itemsize
                               + Cin * Cout * 4)),
    )(xv, xv, wr, beta)

    return out.reshape(N, H, W, Cout).transpose(0, 3, 1, 2)


# speedup vs baseline: 1.0121x; 1.0121x over previous
import jax
import jax.numpy as jnp
from jax.experimental import pallas as pl
from jax.experimental.pallas import tpu as pltpu

_MIB = 1024 * 1024


def _fused_kernel(x1_ref, x2_ref, w_ref, a_ref, b_ref, o_ref):
    cout, k, _ = w_ref.shape
    w2d = w_ref[...].reshape(cout, k * 128)
    half = (k * 128) // 2
    s1 = jnp.sum(x1_ref[0], axis=0, keepdims=True)
    s2 = jnp.sum(x2_ref[0], axis=0, keepdims=True)
    dn = (((1,), (1,)), ((), ()))
    y = (jax.lax.dot_general(s1, w2d[:, :half], dn,
                             preferred_element_type=jnp.float32)
         + jax.lax.dot_general(s2, w2d[:, half:], dn,
                               preferred_element_type=jnp.float32))
    z = jnp.maximum(y * a_ref[...] + b_ref[...], 0.0)
    o_ref[0] = jnp.broadcast_to(z, o_ref.shape[1:]).astype(o_ref.dtype)


def kernel(x, conv_w, bn_gamma, bn_beta, bn_mean, bn_var, eps=1e-5):
    N, Cin, H, W = x.shape
    Cout = conv_w.shape[0]
    HW = H * W

    scale = (bn_gamma.astype(jnp.float32)
             / jnp.sqrt(bn_var.astype(jnp.float32) + eps))
    bias = bn_beta.astype(jnp.float32) - bn_mean.astype(jnp.float32) * scale
    alpha = (scale * (1.0 / HW))[None, :]
    beta = bias[None, :]
    wr = conv_w.reshape(Cout, Cin // 128, 128).astype(jnp.float32)

    xv = jnp.transpose(x, (0, 2, 3, 1)).reshape(N, HW, Cin)
    itemsize = jnp.dtype(x.dtype).itemsize
    ch = Cin // 2

    out = pl.pallas_call(
        _fused_kernel,
        out_shape=jax.ShapeDtypeStruct((N, HW, Cout), x.dtype),
        grid=(N,),
        in_specs=[
            pl.BlockSpec((1, HW, ch), lambda n: (n, 0, 0)),
            pl.BlockSpec((1, HW, ch), lambda n: (n, 0, 1)),
            pl.BlockSpec((Cout, Cin // 128, 128), lambda n: (0, 0, 0)),
            pl.BlockSpec((1, Cout), lambda n: (0, 0)),
            pl.BlockSpec((1, Cout), lambda n: (0, 0)),
        ],
        out_specs=pl.BlockSpec((1, HW, Cout), lambda n: (n, 0, 0)),
        compiler_params=pltpu.CompilerParams(
            dimension_semantics=("parallel",),
            vmem_limit_bytes=48 * _MIB),
        cost_estimate=pl.CostEstimate(
            flops=int(N * Cin * HW + 2 * N * Cin * Cout),
            transcendentals=0,
            bytes_accessed=int(N * Cin * HW * itemsize
                               + N * Cout * HW * itemsize
                               + Cin * Cout * 4)),
    )(xv, xv, wr, alpha, beta)

    return out.reshape(N, H, W, Cout).transpose(0, 3, 1, 2)


# speedup vs baseline: 1.0126x; 1.0004x over previous
import jax
import jax.numpy as jnp
from jax.experimental import pallas as pl
from jax.experimental.pallas import tpu as pltpu

_MIB = 1024 * 1024


def _fused_kernel(x_ref, w_ref, a_ref, b_ref, o_ref):
    cout, k, _ = w_ref.shape
    w2d = w_ref[...].reshape(cout, k * 128)
    s = jnp.sum(x_ref[0], axis=0, keepdims=True)
    y = jax.lax.dot_general(s, w2d,
                            (((1,), (1,)), ((), ())),
                            preferred_element_type=jnp.float32)
    z = jnp.maximum(y * a_ref[...] + b_ref[...], 0.0)
    o_ref[0] = jnp.broadcast_to(z, o_ref.shape[1:]).astype(o_ref.dtype)


def kernel(x, conv_w, bn_gamma, bn_beta, bn_mean, bn_var, eps=1e-5):
    N, Cin, H, W = x.shape
    Cout = conv_w.shape[0]
    HW = H * W

    scale = (bn_gamma.astype(jnp.float32)
             / jnp.sqrt(bn_var.astype(jnp.float32) + eps))
    bias = bn_beta.astype(jnp.float32) - bn_mean.astype(jnp.float32) * scale
    alpha = (scale * (1.0 / HW))[None, :]
    beta = bias[None, :]
    wr = conv_w.reshape(Cout, Cin // 128, 128).astype(jnp.float32)

    xv = jnp.transpose(x, (0, 2, 3, 1)).reshape(N, HW, Cin)
    itemsize = jnp.dtype(x.dtype).itemsize

    out = pl.pallas_call(
        _fused_kernel,
        out_shape=jax.ShapeDtypeStruct((N, HW, Cout), x.dtype),
        grid=(N,),
        in_specs=[
            pl.BlockSpec((1, HW, Cin), lambda n: (n, 0, 0)),
            pl.BlockSpec((Cout, Cin // 128, 128), lambda n: (0, 0, 0)),
            pl.BlockSpec((1, Cout), lambda n: (0, 0)),
            pl.BlockSpec((1, Cout), lambda n: (0, 0)),
        ],
        out_specs=pl.BlockSpec((1, HW, Cout), lambda n: (n, 0, 0)),
        compiler_params=pltpu.CompilerParams(
            dimension_semantics=("parallel",),
            vmem_limit_bytes=48 * _MIB),
        cost_estimate=pl.CostEstimate(
            flops=int(N * Cin * HW + 2 * N * Cin * Cout),
            transcendentals=0,
            bytes_accessed=int(N * Cin * HW * itemsize
                               + N * Cout * HW * itemsize
                               + Cin * Cout * 4)),
    )(xv, wr, alpha, beta)

    return out.reshape(N, H, W, Cout).transpose(0, 3, 1, 2)
